# baseline (speedup 1.0000x reference)
import jax
import jax.numpy as jnp
from jax import lax
from jax.experimental import pallas as pl
from jax.experimental.pallas import tpu as pltpu
from jax.experimental.pallas import tpu_sc as plsc

VOCAB_SZ = 400005
EMBED_SZ = 300
BATCH = 4096
SEQ = 200

_DP = 304
_NC = 2
_NS = 16
_NW = _NC * _NS
_B = BATCH * SEQ
_BPW = _B // _NW
_CHUNK = 200
_NCHUNKS = _BPW // _CHUNK
_NBUF = 2


def _gather_body(idx_hbm, table_hbm, out_hbm, idx_v, rows_v, isems, gsems, osems):
    wid = lax.axis_index("s") * _NC + lax.axis_index("c")
    base = wid * _BPW

    def start_idx(c, buf):
        pltpu.async_copy(idx_hbm.at[pl.ds(base + c * _CHUNK, _CHUNK)],
                         idx_v.at[buf], isems.at[buf])

    def step(c, carry):
        buf = lax.rem(c, _NBUF)
        pltpu.make_async_copy(idx_hbm.at[pl.ds(0, _CHUNK)], idx_v.at[buf],
                              isems.at[buf]).wait()

        @pl.when(c >= _NBUF)
        def _():
            pltpu.make_async_copy(rows_v.at[buf], out_hbm.at[pl.ds(0, _CHUNK)],
                                  osems.at[buf]).wait()
        pltpu.async_copy(table_hbm.at[idx_v.at[buf]], rows_v.at[buf],
                         gsems.at[buf]).wait()

        @pl.when(c + _NBUF < _NCHUNKS)
        def _():
            start_idx(c + _NBUF, buf)
        pltpu.async_copy(rows_v.at[buf], out_hbm.at[pl.ds(base + c * _CHUNK, _CHUNK)],
                         osems.at[buf])
        return carry

    for b in range(_NBUF):
        start_idx(b, b)
    lax.fori_loop(0, _NCHUNKS, step, 0)
    for b in range(_NBUF):
        buf = (_NCHUNKS - 1 - b) % _NBUF
        pltpu.make_async_copy(rows_v.at[buf], out_hbm.at[pl.ds(0, _CHUNK)],
                              osems.at[buf]).wait()


@jax.jit
def kernel(x, embed_weight):
    idx = x.reshape(_B).astype(jnp.int32)
    table_p = jnp.pad(embed_weight, ((0, 0), (0, _DP - EMBED_SZ)))
    mesh = plsc.VectorSubcoreMesh(core_axis_name="c", subcore_axis_name="s")
    out_p = pl.kernel(
        _gather_body,
        out_type=jax.ShapeDtypeStruct((_B, _DP), jnp.float32),
        mesh=mesh,
        scratch_types=[
            pltpu.VMEM((_NBUF, _CHUNK), jnp.int32),
            pltpu.VMEM((_NBUF, _CHUNK, _DP), jnp.float32),
            pltpu.SemaphoreType.DMA((_NBUF,)),
            pltpu.SemaphoreType.DMA((_NBUF,)),
            pltpu.SemaphoreType.DMA((_NBUF,)),
        ],
        compiler_params=pltpu.CompilerParams(use_tc_tiling_on_sc=False),
    )(idx, table_p)
    return out_p[:, :EMBED_SZ].reshape(BATCH, SEQ, EMBED_SZ)
